# baseline (speedup 1.0000x reference)
import functools

import jax
import jax.numpy as jnp
from jax import lax
from jax.experimental import pallas as pl
from jax.experimental.pallas import tpu as pltpu
from jax.experimental.pallas import tpu_sc as plsc

N = 10000
E = 320000
F = 128
H = 128
NCORES = 2
NSUB = 16
CHUNK = 80
EPS = E // NSUB
NCH = EPS // CHUNK
NBUF = 5
assert NCH % NBUF == 0
NSC = 10112
RPS = NSC // NSUB
NPAD = 10240
BLK = 1024
GRID = NPAD // BLK

_sc_mesh = plsc.VectorSubcoreMesh(core_axis_name="c", subcore_axis_name="s")


def _sc_segment_sum(tbl, src_flat, dst_r, width, dtype):
    @functools.partial(
        pl.kernel,
        out_type=jax.ShapeDtypeStruct((NCORES, NPAD, width), dtype),
        mesh=_sc_mesh,
        compiler_params=pltpu.CompilerParams(use_tc_tiling_on_sc=False),
        scratch_types=[
            pltpu.VMEM((EPS,), jnp.int32),
            pltpu.VMEM((NCH, CHUNK), jnp.int32),
        ] + [pltpu.VMEM((CHUNK, width), dtype)] * NBUF + [
            pltpu.VMEM_SHARED((NSC, width), dtype),
            pltpu.SemaphoreType.DMA,
        ] + [pltpu.SemaphoreType.DMA] * (2 * NBUF),
    )
    def seg_sum_kernel(tbl_hbm, e_hbm, d_hbm, out_hbm, sidx, didx, *scr):
        rows = scr[:NBUF]
        agg = scr[NBUF]
        isem = scr[NBUF + 1]
        gsem = scr[NBUF + 2:NBUF + 2 + NBUF]
        ssem = scr[NBUF + 2 + NBUF:]
        cid = lax.axis_index("c")
        sid = lax.axis_index("s")
        ebase = sid * EPS
        pltpu.async_copy(e_hbm.at[0, pl.ds(ebase, EPS)], sidx, isem)
        pltpu.async_copy(d_hbm.at[sid], didx, isem)
        z16 = jnp.zeros((16,), dtype)

        @pl.loop(0, CHUNK)
        def _(r):
            for c in range(width // 16):
                rows[0][r, pl.ds(c * 16, 16)] = z16

        nz = RPS // CHUNK
        rem = RPS - nz * CHUNK

        @pl.loop(0, nz)
        def _(i):
            pltpu.sync_copy(rows[0],
                            agg.at[pl.ds(sid * RPS + i * CHUNK, CHUNK)])

        if rem:
            pltpu.sync_copy(rows[0].at[pl.ds(0, rem)],
                            agg.at[pl.ds(sid * RPS + nz * CHUNK, rem)])
        pltpu.make_async_copy(e_hbm.at[0, pl.ds(ebase, EPS)], sidx,
                              isem).wait()
        pltpu.make_async_copy(d_hbm.at[sid], didx, isem).wait()
        plsc.subcore_barrier()

        tb = tbl_hbm.at[cid]

        def gather(i, b):
            return pltpu.make_async_copy(tb.at[sidx.at[pl.ds(i * CHUNK,
                                                             CHUNK)]],
                                         rows[b], gsem[b])

        def scat(i, b):
            return pltpu.make_async_copy(rows[b], agg.at[didx.at[i]], ssem[b])

        for b in range(NBUF - 1):
            gather(b, b).start()

        @pl.loop(0, NCH // NBUF)
        def _(j):
            for k in range(NBUF):
                i = j * NBUF + k
                bn = (k + NBUF - 1) % NBUF

                @pl.when(i + NBUF - 1 < NCH)
                def _():
                    @pl.when(i >= 1)
                    def _():
                        scat(i - 1, bn).wait()

                    gather(i + NBUF - 1, bn).start()

                gather(i, k).wait()
                scat(i, k).start(add=True)

        for k in range(NBUF):
            scat(NCH - NBUF + k, k).wait()

        plsc.subcore_barrier()
        pltpu.sync_copy(agg.at[pl.ds(sid * RPS, RPS)],
                        out_hbm.at[cid, pl.ds(sid * RPS, RPS)])

    return seg_sum_kernel(tbl, src_flat, dst_r)


def _tc_encode(x, WeT, be, Wl0T, Ws0T, bs0):
    def body(x_ref, weT, be_ref, wlT, wsT, bs_ref, m0_ref, s0b_ref):
        h = jnp.dot(x_ref[...], weT[...], preferred_element_type=jnp.float32)
        h = h + be_ref[...]
        m0 = jnp.dot(h, wlT[...], preferred_element_type=jnp.float32)
        m0_ref[0] = m0[:, :H // 2]
        m0_ref[1] = m0[:, H // 2:]
        s0b_ref[...] = jnp.dot(h, wsT[...],
                               preferred_element_type=jnp.float32) + bs_ref[...]

    w_spec = pl.BlockSpec((H, H), lambda i: (0, 0))
    b_spec = pl.BlockSpec((1, H), lambda i: (0, 0))
    row_spec = pl.BlockSpec((BLK, H), lambda i: (i, 0))
    return pl.pallas_call(
        body,
        grid=(GRID,),
        in_specs=[row_spec, w_spec, b_spec, w_spec, w_spec, b_spec],
        out_specs=[pl.BlockSpec((NCORES, BLK, H // 2), lambda i: (0, i, 0)),
                   row_spec],
        out_shape=[
            jax.ShapeDtypeStruct((NCORES, NPAD, H // 2), jnp.float32),
            jax.ShapeDtypeStruct((NPAD, H), jnp.float32),
        ],
    )(x, WeT, be, Wl0T, Ws0T, bs0)


def _tc_spike_mid(agg0, self0b, Ws1T, bs1):
    def body(a_ref, sb_ref, wsT, bs_ref, s8_ref, s1b_ref, cnt_ref):
        i = pl.program_id(0)
        cur = jnp.concatenate([a_ref[0], a_ref[1]], axis=1) + sb_ref[...]
        row = lax.broadcasted_iota(jnp.int32, (BLK, H), 0) + i * BLK
        s = jnp.where((cur >= 1.0) & (row < N), 1.0, 0.0)
        s8 = s.astype(jnp.uint8)
        s8_ref[0] = s8[:, :H // 2]
        s8_ref[1] = s8[:, H // 2:]
        s1b_ref[...] = jnp.dot(s, wsT[...],
                               preferred_element_type=jnp.float32) + bs_ref[...]

        @pl.when(i == 0)
        def _():
            cnt_ref[...] = jnp.zeros_like(cnt_ref)

        cnt_ref[...] += jnp.sum(s).reshape(1, 1)

    agg_spec = pl.BlockSpec((NCORES, BLK, H // 2), lambda i: (0, i, 0))
    row_spec = pl.BlockSpec((BLK, H), lambda i: (i, 0))
    return pl.pallas_call(
        body,
        grid=(GRID,),
        in_specs=[agg_spec, row_spec, pl.BlockSpec((H, H), lambda i: (0, 0)),
                  pl.BlockSpec((1, H), lambda i: (0, 0))],
        out_specs=[pl.BlockSpec((NCORES, BLK, H // 2), lambda i: (0, i, 0)),
                   row_spec, pl.BlockSpec((1, 1), lambda i: (0, 0))],
        out_shape=[
            jax.ShapeDtypeStruct((NCORES, NPAD, H // 2), jnp.uint8),
            jax.ShapeDtypeStruct((NPAD, H), jnp.float32),
            jax.ShapeDtypeStruct((1, 1), jnp.float32),
        ],
    )(agg0, self0b, Ws1T, bs1)


def _tc_heads(cnt1p, self1b, Wl1Tp, Wn1T, bn1, wn2, bn2, Wc1T, bc1, Wc2Tp,
              bc2p):
    def body(a_ref, sb_ref, wl1T, wn1T, bn1_ref, wn2_ref, bn2_ref, wc1T,
             bc1_ref, wc2T, bc2_ref, h_ref, np_ref, cnt_ref, gf_ref, gl_ref):
        i = pl.program_id(0)
        cntf = jnp.concatenate(
            [((a_ref[c] >> (8 * k)) & 0xFF).astype(jnp.float32)
             for c in range(2) for k in range(4)], axis=1)
        cur = jnp.dot(cntf, wl1T[...],
                      preferred_element_type=jnp.float32) + sb_ref[...]
        row = lax.broadcasted_iota(jnp.int32, (BLK, H), 0) + i * BLK
        h = jnp.where((cur >= 1.0) & (row < N), 1.0, 0.0)
        h_ref[...] = h
        nh = jnp.dot(h, wn1T[...], preferred_element_type=jnp.float32)
        nh = jnp.maximum(nh + bn1_ref[...], 0.0)
        logit = jnp.sum(nh * wn2_ref[...], axis=1, keepdims=True) + bn2_ref[0, 0]
        np_ref[...] = jnp.where(
            logit >= 0.0,
            1.0 / (1.0 + jnp.exp(-logit)),
            jnp.exp(logit) / (1.0 + jnp.exp(logit)),
        )

        @pl.when(i == 0)
        def _():
            cnt_ref[...] = jnp.zeros_like(cnt_ref)
            gf_ref[...] = jnp.zeros_like(gf_ref)

        cnt_ref[...] += jnp.sum(h).reshape(1, 1)
        gf_ref[...] += jnp.sum(h, axis=0, keepdims=True)

        @pl.when(i == GRID - 1)
        def _():
            gf = gf_ref[...] / 10000.0
            z = jnp.dot(gf, wc1T[...], preferred_element_type=jnp.float32)
            z = jnp.maximum(z + bc1_ref[...], 0.0)
            gl_ref[...] = jnp.dot(z, wc2T[...],
                                  preferred_element_type=jnp.float32) + bc2_ref[...]

    agg_spec = pl.BlockSpec((NCORES, BLK, H // 8), lambda i: (0, i, 0))
    row_spec = pl.BlockSpec((BLK, H), lambda i: (i, 0))
    fixed = lambda shape: pl.BlockSpec(shape, lambda i: tuple(0 for _ in shape))
    return pl.pallas_call(
        body,
        grid=(GRID,),
        in_specs=[agg_spec, row_spec, fixed((H, H)),
                  fixed((H, H // 2)), fixed((1, H // 2)),
                  fixed((1, H // 2)), fixed((1, 1)),
                  fixed((H, H // 2)), fixed((1, H // 2)),
                  fixed((H // 2, H)), fixed((1, H))],
        out_specs=[row_spec, pl.BlockSpec((BLK, 1), lambda i: (i, 0)),
                   fixed((1, 1)), fixed((1, H)), fixed((1, H))],
        out_shape=[
            jax.ShapeDtypeStruct((N, H), jnp.float32),
            jax.ShapeDtypeStruct((N, 1), jnp.float32),
            jax.ShapeDtypeStruct((1, 1), jnp.float32),
            jax.ShapeDtypeStruct((1, H), jnp.float32),
            jax.ShapeDtypeStruct((1, H), jnp.float32),
        ],
    )(cnt1p, self1b, Wl1Tp, Wn1T, bn1, wn2, bn2, Wc1T, bc1, Wc2Tp, bc2p)


def kernel(x, edge_index, W_enc, b_enc, W_lin0, W_self0, b_self0,
           W_lin1, W_self1, b_self1, Wn1, bn1, Wn2, bn2,
           Wc1, bc1, Wc2, bc2):
    dst_r = edge_index[1].reshape(NSUB, NCH, CHUNK)
    m0, self0b = _tc_encode(
        x, W_enc.T, b_enc.reshape(1, H), W_lin0.T, W_self0.T,
        b_self0.reshape(1, H))
    agg0 = _sc_segment_sum(m0, edge_index, dst_r, H // 2, jnp.float32)
    s8, self1b, cnt0 = _tc_spike_mid(
        agg0, self0b, W_self1.T, b_self1.reshape(1, H))
    s_pack = jax.lax.bitcast_convert_type(
        s8.reshape(NCORES, NPAD, H // 8, 4), jnp.int32)
    cnt1p = _sc_segment_sum(s_pack, edge_index, dst_r, H // 8, jnp.int32)
    perm = [64 * c + 4 * j + k
            for c in range(2) for k in range(4) for j in range(H // 8)]
    Wl1Tp = W_lin1.T[jnp.array(perm), :]
    hp, npr, cnt1, _gf, gl = _tc_heads(
        cnt1p, self1b, Wl1Tp, Wn1.T, bn1.reshape(1, H // 2), Wn2,
        bn2.reshape(1, 1), Wc1.T, bc1.reshape(1, H // 2),
        jnp.pad(Wc2.T, ((0, 0), (0, H - 2))),
        jnp.pad(bc2, (0, H - 2)).reshape(1, H))

    return (gl[:, :2], npr, hp, cnt0[0, 0], cnt1[0, 0])


# speedup vs baseline: 1.1469x; 1.1469x over previous
import functools

import jax
import jax.numpy as jnp
from jax import lax
from jax.experimental import pallas as pl
from jax.experimental.pallas import tpu as pltpu
from jax.experimental.pallas import tpu_sc as plsc

N = 10000
E = 320000
F = 128
H = 128
NCORES = 2
NSUB = 16
CHUNK = 128
NCHT = E // CHUNK
NSC = 10112
RPS = NSC // NSUB
NPAD = 10240
BLK = 1024
GRID = NPAD // BLK

_sc_mesh = plsc.VectorSubcoreMesh(core_axis_name="c", subcore_axis_name="s")


def _sc_segment_sum(tbl, src_flat, dst_r, width, dtype, feature_split, nbuf):
    nunits = NSUB if feature_split else NSUB * NCORES
    main = NCHT // nunits
    extra = NCHT - main * nunits
    assert main % nbuf == 0

    @functools.partial(
        pl.kernel,
        out_type=jax.ShapeDtypeStruct((NCORES, NPAD, width), dtype),
        mesh=_sc_mesh,
        compiler_params=pltpu.CompilerParams(use_tc_tiling_on_sc=False),
        scratch_types=[
            pltpu.VMEM(((main + 1) * CHUNK,), jnp.int32),
            pltpu.VMEM((main + 1, CHUNK), jnp.int32),
        ] + [pltpu.VMEM((CHUNK, width), dtype)] * nbuf + [
            pltpu.VMEM_SHARED((NSC, width), dtype),
            pltpu.SemaphoreType.DMA,
        ] + [pltpu.SemaphoreType.DMA] * (2 * nbuf),
    )
    def seg_sum_kernel(tbl_hbm, e_hbm, d_hbm, out_hbm, sidx, didx, *scr):
        rows = scr[:nbuf]
        agg = scr[nbuf]
        isem = scr[nbuf + 1]
        gsem = scr[nbuf + 2:nbuf + 2 + nbuf]
        ssem = scr[nbuf + 2 + nbuf:]
        cid = lax.axis_index("c")
        sid = lax.axis_index("s")
        unit = sid if feature_split else sid * NCORES + cid
        has_extra = unit < extra
        pltpu.async_copy(e_hbm.at[0, pl.ds(unit * main * CHUNK, main * CHUNK)],
                         sidx.at[pl.ds(0, main * CHUNK)], isem)
        pltpu.async_copy(d_hbm.at[pl.ds(unit * main, main)],
                         didx.at[pl.ds(0, main)], isem)

        @pl.when(has_extra)
        def _():
            pltpu.async_copy(
                e_hbm.at[0, pl.ds((nunits * main + unit) * CHUNK, CHUNK)],
                sidx.at[pl.ds(main * CHUNK, CHUNK)], isem)
            pltpu.async_copy(d_hbm.at[pl.ds(nunits * main + unit, 1)],
                             didx.at[pl.ds(main, 1)], isem)

        z16 = jnp.zeros((16,), dtype)

        @pl.loop(0, CHUNK)
        def _(r):
            for c in range(width // 16):
                rows[0][r, pl.ds(c * 16, 16)] = z16

        nz = RPS // CHUNK
        rem = RPS - nz * CHUNK

        @pl.loop(0, nz)
        def _(i):
            pltpu.sync_copy(rows[0],
                            agg.at[pl.ds(sid * RPS + i * CHUNK, CHUNK)])

        if rem:
            pltpu.sync_copy(rows[0].at[pl.ds(0, rem)],
                            agg.at[pl.ds(sid * RPS + nz * CHUNK, rem)])

        pltpu.make_async_copy(
            e_hbm.at[0, pl.ds(unit * main * CHUNK, main * CHUNK)],
            sidx.at[pl.ds(0, main * CHUNK)], isem).wait()
        pltpu.make_async_copy(d_hbm.at[pl.ds(unit * main, main)],
                              didx.at[pl.ds(0, main)], isem).wait()

        @pl.when(has_extra)
        def _():
            pltpu.make_async_copy(
                e_hbm.at[0, pl.ds((nunits * main + unit) * CHUNK, CHUNK)],
                sidx.at[pl.ds(main * CHUNK, CHUNK)], isem).wait()
            pltpu.make_async_copy(d_hbm.at[pl.ds(nunits * main + unit, 1)],
                                  didx.at[pl.ds(main, 1)], isem).wait()

        plsc.subcore_barrier()

        tb = tbl_hbm.at[cid] if feature_split else tbl_hbm

        def gather(i, b):
            return pltpu.make_async_copy(tb.at[sidx.at[pl.ds(i * CHUNK,
                                                             CHUNK)]],
                                         rows[b], gsem[b])

        def scat(i, b):
            return pltpu.make_async_copy(rows[b], agg.at[didx.at[i]], ssem[b])

        for b in range(nbuf - 1):
            gather(b, b).start()

        @pl.loop(0, main // nbuf)
        def _(j):
            for k in range(nbuf):
                i = j * nbuf + k
                bn = (k + nbuf - 1) % nbuf

                @pl.when(i + nbuf - 1 < main)
                def _():
                    @pl.when(i >= 1)
                    def _():
                        scat(i - 1, bn).wait()

                    gather(i + nbuf - 1, bn).start()

                gather(i, k).wait()
                scat(i, k).start(add=True)

        for k in range(nbuf):
            scat(main - nbuf + k, k).wait()

        @pl.when(has_extra)
        def _():
            pltpu.sync_copy(tb.at[sidx.at[pl.ds(main * CHUNK, CHUNK)]],
                            rows[0])
            pltpu.sync_copy(rows[0], agg.at[didx.at[main]], add=True)

        plsc.subcore_barrier()
        pltpu.sync_copy(agg.at[pl.ds(sid * RPS, RPS)],
                        out_hbm.at[cid, pl.ds(sid * RPS, RPS)])

    return seg_sum_kernel(tbl, src_flat, dst_r)


def _tc_encode(x, WeT, be, Wl0T, Ws0T, bs0):
    def body(x_ref, weT, be_ref, wlT, wsT, bs_ref, m0_ref, s0b_ref):
        h = jnp.dot(x_ref[...], weT[...], preferred_element_type=jnp.float32)
        h = h + be_ref[...]
        m0 = jnp.dot(h, wlT[...], preferred_element_type=jnp.float32)
        m0_ref[0] = m0[:, :H // 2]
        m0_ref[1] = m0[:, H // 2:]
        s0b_ref[...] = jnp.dot(h, wsT[...],
                               preferred_element_type=jnp.float32) + bs_ref[...]

    w_spec = pl.BlockSpec((H, H), lambda i: (0, 0))
    b_spec = pl.BlockSpec((1, H), lambda i: (0, 0))
    row_spec = pl.BlockSpec((BLK, H), lambda i: (i, 0))
    return pl.pallas_call(
        body,
        grid=(GRID,),
        in_specs=[row_spec, w_spec, b_spec, w_spec, w_spec, b_spec],
        out_specs=[pl.BlockSpec((NCORES, BLK, H // 2), lambda i: (0, i, 0)),
                   row_spec],
        out_shape=[
            jax.ShapeDtypeStruct((NCORES, NPAD, H // 2), jnp.float32),
            jax.ShapeDtypeStruct((NPAD, H), jnp.float32),
        ],
    )(x, WeT, be, Wl0T, Ws0T, bs0)


def _tc_spike_mid(agg0, self0b, Ws1T, bs1):
    def body(a_ref, sb_ref, wsT, bs_ref, s8_ref, s1b_ref, cnt_ref):
        i = pl.program_id(0)
        cur = jnp.concatenate([a_ref[0], a_ref[1]], axis=1) + sb_ref[...]
        row = lax.broadcasted_iota(jnp.int32, (BLK, H), 0) + i * BLK
        s = jnp.where((cur >= 1.0) & (row < N), 1.0, 0.0)
        s8_ref[...] = s.astype(jnp.uint8)
        s1b_ref[...] = jnp.dot(s, wsT[...],
                               preferred_element_type=jnp.float32) + bs_ref[...]

        @pl.when(i == 0)
        def _():
            cnt_ref[...] = jnp.zeros_like(cnt_ref)

        cnt_ref[...] += jnp.sum(s).reshape(1, 1)

    agg_spec = pl.BlockSpec((NCORES, BLK, H // 2), lambda i: (0, i, 0))
    row_spec = pl.BlockSpec((BLK, H), lambda i: (i, 0))
    return pl.pallas_call(
        body,
        grid=(GRID,),
        in_specs=[agg_spec, row_spec, pl.BlockSpec((H, H), lambda i: (0, 0)),
                  pl.BlockSpec((1, H), lambda i: (0, 0))],
        out_specs=[row_spec, row_spec, pl.BlockSpec((1, 1), lambda i: (0, 0))],
        out_shape=[
            jax.ShapeDtypeStruct((NPAD, H), jnp.uint8),
            jax.ShapeDtypeStruct((NPAD, H), jnp.float32),
            jax.ShapeDtypeStruct((1, 1), jnp.float32),
        ],
    )(agg0, self0b, Ws1T, bs1)


def _tc_heads(cnt1p, self1b, Wl1Tp, Wn1T, bn1, wn2, bn2, Wc1T, bc1, Wc2Tp,
              bc2p):
    def body(a_ref, sb_ref, wl1T, wn1T, bn1_ref, wn2_ref, bn2_ref, wc1T,
             bc1_ref, wc2T, bc2_ref, h_ref, np_ref, cnt_ref, gf_ref, gl_ref):
        i = pl.program_id(0)
        wa, wb = a_ref[0], a_ref[1]
        cntf = jnp.concatenate(
            [(((wa >> (8 * k)) & 0xFF) + ((wb >> (8 * k)) & 0xFF)
              ).astype(jnp.float32) for k in range(4)], axis=1)
        cur = jnp.dot(cntf, wl1T[...],
                      preferred_element_type=jnp.float32) + sb_ref[...]
        row = lax.broadcasted_iota(jnp.int32, (BLK, H), 0) + i * BLK
        h = jnp.where((cur >= 1.0) & (row < N), 1.0, 0.0)
        h_ref[...] = h
        nh = jnp.dot(h, wn1T[...], preferred_element_type=jnp.float32)
        nh = jnp.maximum(nh + bn1_ref[...], 0.0)
        logit = jnp.sum(nh * wn2_ref[...], axis=1, keepdims=True) + bn2_ref[0, 0]
        np_ref[...] = jnp.where(
            logit >= 0.0,
            1.0 / (1.0 + jnp.exp(-logit)),
            jnp.exp(logit) / (1.0 + jnp.exp(logit)),
        )

        @pl.when(i == 0)
        def _():
            cnt_ref[...] = jnp.zeros_like(cnt_ref)
            gf_ref[...] = jnp.zeros_like(gf_ref)

        cnt_ref[...] += jnp.sum(h).reshape(1, 1)
        gf_ref[...] += jnp.sum(h, axis=0, keepdims=True)

        @pl.when(i == GRID - 1)
        def _():
            gf = gf_ref[...] / 10000.0
            z = jnp.dot(gf, wc1T[...], preferred_element_type=jnp.float32)
            z = jnp.maximum(z + bc1_ref[...], 0.0)
            gl_ref[...] = jnp.dot(z, wc2T[...],
                                  preferred_element_type=jnp.float32) + bc2_ref[...]

    agg_spec = pl.BlockSpec((NCORES, BLK, H // 4), lambda i: (0, i, 0))
    row_spec = pl.BlockSpec((BLK, H), lambda i: (i, 0))
    fixed = lambda shape: pl.BlockSpec(shape, lambda i: tuple(0 for _ in shape))
    return pl.pallas_call(
        body,
        grid=(GRID,),
        in_specs=[agg_spec, row_spec, fixed((H, H)),
                  fixed((H, H // 2)), fixed((1, H // 2)),
                  fixed((1, H // 2)), fixed((1, 1)),
                  fixed((H, H // 2)), fixed((1, H // 2)),
                  fixed((H // 2, H)), fixed((1, H))],
        out_specs=[row_spec, pl.BlockSpec((BLK, 1), lambda i: (i, 0)),
                   fixed((1, 1)), fixed((1, H)), fixed((1, H))],
        out_shape=[
            jax.ShapeDtypeStruct((N, H), jnp.float32),
            jax.ShapeDtypeStruct((N, 1), jnp.float32),
            jax.ShapeDtypeStruct((1, 1), jnp.float32),
            jax.ShapeDtypeStruct((1, H), jnp.float32),
            jax.ShapeDtypeStruct((1, H), jnp.float32),
        ],
    )(cnt1p, self1b, Wl1Tp, Wn1T, bn1, wn2, bn2, Wc1T, bc1, Wc2Tp, bc2p)


def kernel(x, edge_index, W_enc, b_enc, W_lin0, W_self0, b_self0,
           W_lin1, W_self1, b_self1, Wn1, bn1, Wn2, bn2,
           Wc1, bc1, Wc2, bc2):
    dst_r = edge_index[1].reshape(NCHT, CHUNK)
    m0, self0b = _tc_encode(
        x, W_enc.T, b_enc.reshape(1, H), W_lin0.T, W_self0.T,
        b_self0.reshape(1, H))
    agg0 = _sc_segment_sum(m0, edge_index, dst_r, H // 2, jnp.float32,
                           feature_split=True, nbuf=4)
    s8, self1b, cnt0 = _tc_spike_mid(
        agg0, self0b, W_self1.T, b_self1.reshape(1, H))
    s_pack = jax.lax.bitcast_convert_type(
        s8.reshape(NPAD, H // 4, 4), jnp.int32)
    cnt1p = _sc_segment_sum(s_pack, edge_index, dst_r, H // 4, jnp.int32,
                            feature_split=False, nbuf=6)
    perm = [4 * j + k for k in range(4) for j in range(H // 4)]
    Wl1Tp = W_lin1.T[jnp.array(perm), :]
    hp, npr, cnt1, _gf, gl = _tc_heads(
        cnt1p, self1b, Wl1Tp, Wn1.T, bn1.reshape(1, H // 2), Wn2,
        bn2.reshape(1, 1), Wc1.T, bc1.reshape(1, H // 2),
        jnp.pad(Wc2.T, ((0, 0), (0, H - 2))),
        jnp.pad(bc2, (0, H - 2)).reshape(1, H))

    return (gl[:, :2], npr, hp, cnt0[0, 0], cnt1[0, 0])


# speedup vs baseline: 1.1985x; 1.0449x over previous
import functools

import jax
import jax.numpy as jnp
from jax import lax
from jax.experimental import pallas as pl
from jax.experimental.pallas import tpu as pltpu
from jax.experimental.pallas import tpu_sc as plsc

N = 10000
E = 320000
F = 128
H = 128
NCORES = 2
NSUB = 16
CHUNK = 128
NCHT = E // CHUNK
NSC = 10112
RPS = NSC // NSUB
NPAD = 10240
BLK = 1024
GRID = NPAD // BLK

_sc_mesh = plsc.VectorSubcoreMesh(core_axis_name="c", subcore_axis_name="s")


def _sc_segment_sum(tbl, e3, width, dtype, feature_split, nbuf):
    nunits = NSUB if feature_split else NSUB * NCORES
    main = NCHT // nunits
    extra = NCHT - main * nunits
    assert main % nbuf == 0

    @functools.partial(
        pl.kernel,
        out_type=jax.ShapeDtypeStruct((NCORES, NPAD, width), dtype),
        mesh=_sc_mesh,
        compiler_params=pltpu.CompilerParams(use_tc_tiling_on_sc=False),
        scratch_types=[
            pltpu.VMEM((main + 1, CHUNK), jnp.int32),
            pltpu.VMEM((main + 1, CHUNK), jnp.int32),
        ] + [pltpu.VMEM((CHUNK, width), dtype)] * nbuf + [
            pltpu.VMEM_SHARED((NSC, width), dtype),
            pltpu.SemaphoreType.DMA,
        ] + [pltpu.SemaphoreType.DMA] * (2 * nbuf),
    )
    def seg_sum_kernel(tbl_hbm, e_hbm, out_hbm, sidx, didx, *scr):
        rows = scr[:nbuf]
        agg = scr[nbuf]
        isem = scr[nbuf + 1]
        gsem = scr[nbuf + 2:nbuf + 2 + nbuf]
        ssem = scr[nbuf + 2 + nbuf:]
        cid = lax.axis_index("c")
        sid = lax.axis_index("s")
        unit = sid if feature_split else sid * NCORES + cid
        has_extra = unit < extra
        pltpu.async_copy(e_hbm.at[0, pl.ds(unit * main, main)],
                         sidx.at[pl.ds(0, main)], isem)
        pltpu.async_copy(e_hbm.at[1, pl.ds(unit * main, main)],
                         didx.at[pl.ds(0, main)], isem)

        @pl.when(has_extra)
        def _():
            pltpu.async_copy(e_hbm.at[0, pl.ds(nunits * main + unit, 1)],
                             sidx.at[pl.ds(main, 1)], isem)
            pltpu.async_copy(e_hbm.at[1, pl.ds(nunits * main + unit, 1)],
                             didx.at[pl.ds(main, 1)], isem)

        z16 = jnp.zeros((16,), dtype)

        @pl.loop(0, CHUNK)
        def _(r):
            for c in range(width // 16):
                rows[0][r, pl.ds(c * 16, 16)] = z16

        nz = RPS // CHUNK
        rem = RPS - nz * CHUNK

        @pl.loop(0, nz)
        def _(i):
            pltpu.sync_copy(rows[0],
                            agg.at[pl.ds(sid * RPS + i * CHUNK, CHUNK)])

        if rem:
            pltpu.sync_copy(rows[0].at[pl.ds(0, rem)],
                            agg.at[pl.ds(sid * RPS + nz * CHUNK, rem)])

        pltpu.make_async_copy(e_hbm.at[0, pl.ds(unit * main, main)],
                              sidx.at[pl.ds(0, main)], isem).wait()
        pltpu.make_async_copy(e_hbm.at[1, pl.ds(unit * main, main)],
                              didx.at[pl.ds(0, main)], isem).wait()

        @pl.when(has_extra)
        def _():
            pltpu.make_async_copy(e_hbm.at[0, pl.ds(nunits * main + unit, 1)],
                                  sidx.at[pl.ds(main, 1)], isem).wait()
            pltpu.make_async_copy(e_hbm.at[1, pl.ds(nunits * main + unit, 1)],
                                  didx.at[pl.ds(main, 1)], isem).wait()

        plsc.subcore_barrier()

        tb = tbl_hbm.at[cid] if feature_split else tbl_hbm

        def gather(i, b):
            return pltpu.make_async_copy(tb.at[sidx.at[i]], rows[b], gsem[b])

        def scat(i, b):
            return pltpu.make_async_copy(rows[b], agg.at[didx.at[i]], ssem[b])

        for b in range(nbuf - 1):
            gather(b, b).start()

        @pl.loop(0, main // nbuf)
        def _(j):
            for k in range(nbuf):
                i = j * nbuf + k
                bn = (k + nbuf - 1) % nbuf

                @pl.when(i + nbuf - 1 < main)
                def _():
                    @pl.when(i >= 1)
                    def _():
                        scat(i - 1, bn).wait()

                    gather(i + nbuf - 1, bn).start()

                gather(i, k).wait()
                scat(i, k).start(add=True)

        for k in range(nbuf):
            scat(main - nbuf + k, k).wait()

        @pl.when(has_extra)
        def _():
            pltpu.sync_copy(tb.at[sidx.at[main]], rows[0])
            pltpu.sync_copy(rows[0], agg.at[didx.at[main]], add=True)

        plsc.subcore_barrier()
        pltpu.sync_copy(agg.at[pl.ds(sid * RPS, RPS)],
                        out_hbm.at[cid, pl.ds(sid * RPS, RPS)])

    return seg_sum_kernel(tbl, e3)


def _tc_encode(x, WeT, be, Wl0T, Ws0T, bs0):
    def body(x_ref, weT, be_ref, wlT, wsT, bs_ref, m0_ref, s0b_ref):
        h = jnp.dot(x_ref[...], weT[...], preferred_element_type=jnp.float32)
        h = h + be_ref[...]
        m0 = jnp.dot(h, wlT[...], preferred_element_type=jnp.float32)
        m0_ref[0] = m0[:, :H // 2]
        m0_ref[1] = m0[:, H // 2:]
        s0b_ref[...] = jnp.dot(h, wsT[...],
                               preferred_element_type=jnp.float32) + bs_ref[...]

    w_spec = pl.BlockSpec((H, H), lambda i: (0, 0))
    b_spec = pl.BlockSpec((1, H), lambda i: (0, 0))
    row_spec = pl.BlockSpec((BLK, H), lambda i: (i, 0))
    return pl.pallas_call(
        body,
        grid=(GRID,),
        in_specs=[row_spec, w_spec, b_spec, w_spec, w_spec, b_spec],
        out_specs=[pl.BlockSpec((NCORES, BLK, H // 2), lambda i: (0, i, 0)),
                   row_spec],
        out_shape=[
            jax.ShapeDtypeStruct((NCORES, NPAD, H // 2), jnp.float32),
            jax.ShapeDtypeStruct((NPAD, H), jnp.float32),
        ],
    )(x, WeT, be, Wl0T, Ws0T, bs0)


def _tc_spike_mid(agg0, self0b, Ws1T, bs1):
    def body(a_ref, sb_ref, wsT, bs_ref, s8_ref, s1b_ref, cnt_ref):
        i = pl.program_id(0)
        cur = jnp.concatenate([a_ref[0], a_ref[1]], axis=1) + sb_ref[...]
        row = lax.broadcasted_iota(jnp.int32, (BLK, H), 0) + i * BLK
        s = jnp.where((cur >= 1.0) & (row < N), 1.0, 0.0)
        s8_ref[...] = s.astype(jnp.uint8)
        s1b_ref[...] = jnp.dot(s, wsT[...],
                               preferred_element_type=jnp.float32) + bs_ref[...]

        @pl.when(i == 0)
        def _():
            cnt_ref[...] = jnp.zeros_like(cnt_ref)

        cnt_ref[...] += jnp.sum(s).reshape(1, 1)

    agg_spec = pl.BlockSpec((NCORES, BLK, H // 2), lambda i: (0, i, 0))
    row_spec = pl.BlockSpec((BLK, H), lambda i: (i, 0))
    return pl.pallas_call(
        body,
        grid=(GRID,),
        in_specs=[agg_spec, row_spec, pl.BlockSpec((H, H), lambda i: (0, 0)),
                  pl.BlockSpec((1, H), lambda i: (0, 0))],
        out_specs=[row_spec, row_spec, pl.BlockSpec((1, 1), lambda i: (0, 0))],
        out_shape=[
            jax.ShapeDtypeStruct((NPAD, H), jnp.uint8),
            jax.ShapeDtypeStruct((NPAD, H), jnp.float32),
            jax.ShapeDtypeStruct((1, 1), jnp.float32),
        ],
    )(agg0, self0b, Ws1T, bs1)


def _tc_heads(cnt1p, self1b, Wl1Tp, Wn1T, bn1, wn2, bn2, Wc1T, bc1, Wc2Tp,
              bc2p):
    def body(a_ref, sb_ref, wl1T, wn1T, bn1_ref, wn2_ref, bn2_ref, wc1T,
             bc1_ref, wc2T, bc2_ref, h_ref, np_ref, cnt_ref, gf_ref, gl_ref):
        i = pl.program_id(0)
        wa, wb = a_ref[0], a_ref[1]
        cntf = jnp.concatenate(
            [(((wa >> (8 * k)) & 0xFF) + ((wb >> (8 * k)) & 0xFF)
              ).astype(jnp.float32) for k in range(4)], axis=1)
        cur = jnp.dot(cntf, wl1T[...],
                      preferred_element_type=jnp.float32) + sb_ref[...]
        row = lax.broadcasted_iota(jnp.int32, (BLK, H), 0) + i * BLK
        h = jnp.where((cur >= 1.0) & (row < N), 1.0, 0.0)
        h_ref[...] = h
        nh = jnp.dot(h, wn1T[...], preferred_element_type=jnp.float32)
        nh = jnp.maximum(nh + bn1_ref[...], 0.0)
        logit = jnp.sum(nh * wn2_ref[...], axis=1, keepdims=True) + bn2_ref[0, 0]
        np_ref[...] = jnp.where(
            logit >= 0.0,
            1.0 / (1.0 + jnp.exp(-logit)),
            jnp.exp(logit) / (1.0 + jnp.exp(logit)),
        )

        @pl.when(i == 0)
        def _():
            cnt_ref[...] = jnp.zeros_like(cnt_ref)
            gf_ref[...] = jnp.zeros_like(gf_ref)

        cnt_ref[...] += jnp.sum(h).reshape(1, 1)
        gf_ref[...] += jnp.sum(h, axis=0, keepdims=True)

        @pl.when(i == GRID - 1)
        def _():
            gf = gf_ref[...] / 10000.0
            z = jnp.dot(gf, wc1T[...], preferred_element_type=jnp.float32)
            z = jnp.maximum(z + bc1_ref[...], 0.0)
            gl_ref[...] = jnp.dot(z, wc2T[...],
                                  preferred_element_type=jnp.float32) + bc2_ref[...]

    agg_spec = pl.BlockSpec((NCORES, BLK, H // 4), lambda i: (0, i, 0))
    row_spec = pl.BlockSpec((BLK, H), lambda i: (i, 0))
    fixed = lambda shape: pl.BlockSpec(shape, lambda i: tuple(0 for _ in shape))
    return pl.pallas_call(
        body,
        grid=(GRID,),
        in_specs=[agg_spec, row_spec, fixed((H, H)),
                  fixed((H, H // 2)), fixed((1, H // 2)),
                  fixed((1, H // 2)), fixed((1, 1)),
                  fixed((H, H // 2)), fixed((1, H // 2)),
                  fixed((H // 2, H)), fixed((1, H))],
        out_specs=[row_spec, pl.BlockSpec((BLK, 1), lambda i: (i, 0)),
                   fixed((1, 1)), fixed((1, H)), fixed((1, H))],
        out_shape=[
            jax.ShapeDtypeStruct((N, H), jnp.float32),
            jax.ShapeDtypeStruct((N, 1), jnp.float32),
            jax.ShapeDtypeStruct((1, 1), jnp.float32),
            jax.ShapeDtypeStruct((1, H), jnp.float32),
            jax.ShapeDtypeStruct((1, H), jnp.float32),
        ],
    )(cnt1p, self1b, Wl1Tp, Wn1T, bn1, wn2, bn2, Wc1T, bc1, Wc2Tp, bc2p)


def kernel(x, edge_index, W_enc, b_enc, W_lin0, W_self0, b_self0,
           W_lin1, W_self1, b_self1, Wn1, bn1, Wn2, bn2,
           Wc1, bc1, Wc2, bc2):
    e3 = edge_index.reshape(2, NCHT, CHUNK)
    m0, self0b = _tc_encode(
        x, W_enc.T, b_enc.reshape(1, H), W_lin0.T, W_self0.T,
        b_self0.reshape(1, H))
    agg0 = _sc_segment_sum(m0, e3, H // 2, jnp.float32,
                           feature_split=True, nbuf=4)
    s8, self1b, cnt0 = _tc_spike_mid(
        agg0, self0b, W_self1.T, b_self1.reshape(1, H))
    s_pack = jax.lax.bitcast_convert_type(
        s8.reshape(NPAD, H // 4, 4), jnp.int32)
    cnt1p = _sc_segment_sum(s_pack, e3, H // 4, jnp.int32,
                            feature_split=False, nbuf=6)
    perm = [4 * j + k for k in range(4) for j in range(H // 4)]
    Wl1Tp = W_lin1.T[jnp.array(perm), :]
    hp, npr, cnt1, _gf, gl = _tc_heads(
        cnt1p, self1b, Wl1Tp, Wn1.T, bn1.reshape(1, H // 2), Wn2,
        bn2.reshape(1, 1), Wc1.T, bc1.reshape(1, H // 2),
        jnp.pad(Wc2.T, ((0, 0), (0, H - 2))),
        jnp.pad(bc2, (0, H - 2)).reshape(1, H))

    return (gl[:, :2], npr, hp, cnt0[0, 0], cnt1[0, 0])


# speedup vs baseline: 1.2393x; 1.0341x over previous
import functools

import jax
import jax.numpy as jnp
from jax import lax
from jax.experimental import pallas as pl
from jax.experimental.pallas import tpu as pltpu
from jax.experimental.pallas import tpu_sc as plsc

N = 10000
E = 320000
F = 128
H = 128
NCORES = 2
NSUB = 16
CHUNK = 128
NCHT = E // CHUNK
NSC = 10112
RPS = NSC // NSUB
NPAD = 10240
BLK = 2048
GRID = NPAD // BLK

_sc_mesh = plsc.VectorSubcoreMesh(core_axis_name="c", subcore_axis_name="s")


def _sc_segment_sum(tbl, e3, width, dtype, feature_split, nbuf):
    nunits = NSUB if feature_split else NSUB * NCORES
    main = NCHT // nunits
    extra = NCHT - main * nunits
    assert main % nbuf == 0

    @functools.partial(
        pl.kernel,
        out_type=jax.ShapeDtypeStruct((NCORES, NPAD, width), dtype),
        mesh=_sc_mesh,
        compiler_params=pltpu.CompilerParams(use_tc_tiling_on_sc=False),
        scratch_types=[
            pltpu.VMEM((main + 1, CHUNK), jnp.int32),
            pltpu.VMEM((main + 1, CHUNK), jnp.int32),
        ] + [pltpu.VMEM((CHUNK, width), dtype)] * nbuf + [
            pltpu.VMEM_SHARED((NSC, width), dtype),
            pltpu.SemaphoreType.DMA,
        ] + [pltpu.SemaphoreType.DMA] * (2 * nbuf),
    )
    def seg_sum_kernel(tbl_hbm, e_hbm, out_hbm, sidx, didx, *scr):
        rows = scr[:nbuf]
        agg = scr[nbuf]
        isem = scr[nbuf + 1]
        gsem = scr[nbuf + 2:nbuf + 2 + nbuf]
        ssem = scr[nbuf + 2 + nbuf:]
        cid = lax.axis_index("c")
        sid = lax.axis_index("s")
        unit = sid if feature_split else sid * NCORES + cid
        has_extra = unit < extra
        pltpu.async_copy(e_hbm.at[0, pl.ds(unit * main, main)],
                         sidx.at[pl.ds(0, main)], isem)
        pltpu.async_copy(e_hbm.at[1, pl.ds(unit * main, main)],
                         didx.at[pl.ds(0, main)], isem)

        @pl.when(has_extra)
        def _():
            pltpu.async_copy(e_hbm.at[0, pl.ds(nunits * main + unit, 1)],
                             sidx.at[pl.ds(main, 1)], isem)
            pltpu.async_copy(e_hbm.at[1, pl.ds(nunits * main + unit, 1)],
                             didx.at[pl.ds(main, 1)], isem)

        z16 = jnp.zeros((16,), dtype)

        @pl.loop(0, CHUNK)
        def _(r):
            for c in range(width // 16):
                rows[0][r, pl.ds(c * 16, 16)] = z16

        nz = RPS // CHUNK
        rem = RPS - nz * CHUNK

        @pl.loop(0, nz)
        def _(i):
            pltpu.sync_copy(rows[0],
                            agg.at[pl.ds(sid * RPS + i * CHUNK, CHUNK)])

        if rem:
            pltpu.sync_copy(rows[0].at[pl.ds(0, rem)],
                            agg.at[pl.ds(sid * RPS + nz * CHUNK, rem)])

        pltpu.make_async_copy(e_hbm.at[0, pl.ds(unit * main, main)],
                              sidx.at[pl.ds(0, main)], isem).wait()
        pltpu.make_async_copy(e_hbm.at[1, pl.ds(unit * main, main)],
                              didx.at[pl.ds(0, main)], isem).wait()

        @pl.when(has_extra)
        def _():
            pltpu.make_async_copy(e_hbm.at[0, pl.ds(nunits * main + unit, 1)],
                                  sidx.at[pl.ds(main, 1)], isem).wait()
            pltpu.make_async_copy(e_hbm.at[1, pl.ds(nunits * main + unit, 1)],
                                  didx.at[pl.ds(main, 1)], isem).wait()

        plsc.subcore_barrier()

        tb = tbl_hbm.at[cid] if feature_split else tbl_hbm

        def gather(i, b):
            return pltpu.make_async_copy(tb.at[sidx.at[i]], rows[b], gsem[b])

        def scat(i, b):
            return pltpu.make_async_copy(rows[b], agg.at[didx.at[i]], ssem[b])

        for b in range(nbuf - 1):
            gather(b, b).start()

        @pl.loop(0, main // nbuf)
        def _(j):
            for k in range(nbuf):
                i = j * nbuf + k
                bn = (k + nbuf - 1) % nbuf

                @pl.when(i + nbuf - 1 < main)
                def _():
                    @pl.when(i >= 1)
                    def _():
                        scat(i - 1, bn).wait()

                    gather(i + nbuf - 1, bn).start()

                gather(i, k).wait()
                scat(i, k).start(add=True)

        for k in range(nbuf):
            scat(main - nbuf + k, k).wait()

        @pl.when(has_extra)
        def _():
            pltpu.sync_copy(tb.at[sidx.at[main]], rows[0])
            pltpu.sync_copy(rows[0], agg.at[didx.at[main]], add=True)

        plsc.subcore_barrier()
        pltpu.sync_copy(agg.at[pl.ds(sid * RPS, RPS)],
                        out_hbm.at[cid, pl.ds(sid * RPS, RPS)])

    return seg_sum_kernel(tbl, e3)


def _tc_encode(x, WeT, be, Wl0T, Ws0T, bs0):
    def body(x_ref, weT, be_ref, wlT, wsT, bs_ref, m0_ref, s0b_ref):
        h = jnp.dot(x_ref[...], weT[...], preferred_element_type=jnp.float32)
        h = h + be_ref[...]
        m0 = jnp.dot(h, wlT[...], preferred_element_type=jnp.float32)
        m0_ref[0] = m0[:, :H // 2]
        m0_ref[1] = m0[:, H // 2:]
        s0b_ref[...] = jnp.dot(h, wsT[...],
                               preferred_element_type=jnp.float32) + bs_ref[...]

    w_spec = pl.BlockSpec((H, H), lambda i: (0, 0))
    b_spec = pl.BlockSpec((1, H), lambda i: (0, 0))
    row_spec = pl.BlockSpec((BLK, H), lambda i: (i, 0))
    return pl.pallas_call(
        body,
        grid=(GRID,),
        in_specs=[row_spec, w_spec, b_spec, w_spec, w_spec, b_spec],
        out_specs=[pl.BlockSpec((NCORES, BLK, H // 2), lambda i: (0, i, 0)),
                   row_spec],
        out_shape=[
            jax.ShapeDtypeStruct((NCORES, NPAD, H // 2), jnp.float32),
            jax.ShapeDtypeStruct((NPAD, H), jnp.float32),
        ],
    )(x, WeT, be, Wl0T, Ws0T, bs0)


def _tc_spike_mid(agg0, self0b, Ws1T, bs1):
    def body(a_ref, sb_ref, wsT, bs_ref, s8_ref, s1b_ref, cnt_ref):
        i = pl.program_id(0)
        cur = jnp.concatenate([a_ref[0], a_ref[1]], axis=1) + sb_ref[...]
        row = lax.broadcasted_iota(jnp.int32, (BLK, H), 0) + i * BLK
        s = jnp.where((cur >= 1.0) & (row < N), 1.0, 0.0)
        s8_ref[...] = s.astype(jnp.uint8)
        s1b_ref[...] = jnp.dot(s, wsT[...],
                               preferred_element_type=jnp.float32) + bs_ref[...]

        @pl.when(i == 0)
        def _():
            cnt_ref[...] = jnp.zeros_like(cnt_ref)

        cnt_ref[...] += jnp.sum(s).reshape(1, 1)

    agg_spec = pl.BlockSpec((NCORES, BLK, H // 2), lambda i: (0, i, 0))
    row_spec = pl.BlockSpec((BLK, H), lambda i: (i, 0))
    return pl.pallas_call(
        body,
        grid=(GRID,),
        in_specs=[agg_spec, row_spec, pl.BlockSpec((H, H), lambda i: (0, 0)),
                  pl.BlockSpec((1, H), lambda i: (0, 0))],
        out_specs=[row_spec, row_spec, pl.BlockSpec((1, 1), lambda i: (0, 0))],
        out_shape=[
            jax.ShapeDtypeStruct((NPAD, H), jnp.uint8),
            jax.ShapeDtypeStruct((NPAD, H), jnp.float32),
            jax.ShapeDtypeStruct((1, 1), jnp.float32),
        ],
    )(agg0, self0b, Ws1T, bs1)


def _tc_heads(cnt1p, self1b, Wl1Tp, Wn1T, bn1, wn2, bn2, Wc1T, bc1, Wc2Tp,
              bc2p):
    def body(a_ref, sb_ref, wl1T, wn1T, bn1_ref, wn2_ref, bn2_ref, wc1T,
             bc1_ref, wc2T, bc2_ref, h_ref, np_ref, cnt_ref, gf_ref, gl_ref):
        i = pl.program_id(0)
        wa, wb = a_ref[0], a_ref[1]
        cntf = jnp.concatenate(
            [(((wa >> (8 * k)) & 0xFF) + ((wb >> (8 * k)) & 0xFF)
              ).astype(jnp.float32) for k in range(4)], axis=1)
        cur = jnp.dot(cntf, wl1T[...],
                      preferred_element_type=jnp.float32) + sb_ref[...]
        row = lax.broadcasted_iota(jnp.int32, (BLK, H), 0) + i * BLK
        h = jnp.where((cur >= 1.0) & (row < N), 1.0, 0.0)
        h_ref[...] = h
        nh = jnp.dot(h, wn1T[...], preferred_element_type=jnp.float32)
        nh = jnp.maximum(nh + bn1_ref[...], 0.0)
        logit = jnp.sum(nh * wn2_ref[...], axis=1, keepdims=True) + bn2_ref[0, 0]
        np_ref[...] = jnp.where(
            logit >= 0.0,
            1.0 / (1.0 + jnp.exp(-logit)),
            jnp.exp(logit) / (1.0 + jnp.exp(logit)),
        )

        @pl.when(i == 0)
        def _():
            cnt_ref[...] = jnp.zeros_like(cnt_ref)
            gf_ref[...] = jnp.zeros_like(gf_ref)

        cnt_ref[...] += jnp.sum(h).reshape(1, 1)
        gf_ref[...] += jnp.sum(h, axis=0, keepdims=True)

        @pl.when(i == GRID - 1)
        def _():
            gf = gf_ref[...] / 10000.0
            z = jnp.dot(gf, wc1T[...], preferred_element_type=jnp.float32)
            z = jnp.maximum(z + bc1_ref[...], 0.0)
            gl_ref[...] = jnp.dot(z, wc2T[...],
                                  preferred_element_type=jnp.float32) + bc2_ref[...]

    agg_spec = pl.BlockSpec((NCORES, BLK, H // 4), lambda i: (0, i, 0))
    row_spec = pl.BlockSpec((BLK, H), lambda i: (i, 0))
    fixed = lambda shape: pl.BlockSpec(shape, lambda i: tuple(0 for _ in shape))
    return pl.pallas_call(
        body,
        grid=(GRID,),
        in_specs=[agg_spec, row_spec, fixed((H, H)),
                  fixed((H, H // 2)), fixed((1, H // 2)),
                  fixed((1, H // 2)), fixed((1, 1)),
                  fixed((H, H // 2)), fixed((1, H // 2)),
                  fixed((H // 2, H)), fixed((1, H))],
        out_specs=[row_spec, pl.BlockSpec((BLK, 1), lambda i: (i, 0)),
                   fixed((1, 1)), fixed((1, H)), fixed((1, H))],
        out_shape=[
            jax.ShapeDtypeStruct((N, H), jnp.float32),
            jax.ShapeDtypeStruct((N, 1), jnp.float32),
            jax.ShapeDtypeStruct((1, 1), jnp.float32),
            jax.ShapeDtypeStruct((1, H), jnp.float32),
            jax.ShapeDtypeStruct((1, H), jnp.float32),
        ],
    )(cnt1p, self1b, Wl1Tp, Wn1T, bn1, wn2, bn2, Wc1T, bc1, Wc2Tp, bc2p)


def kernel(x, edge_index, W_enc, b_enc, W_lin0, W_self0, b_self0,
           W_lin1, W_self1, b_self1, Wn1, bn1, Wn2, bn2,
           Wc1, bc1, Wc2, bc2):
    e3 = edge_index.reshape(2, NCHT, CHUNK)
    m0, self0b = _tc_encode(
        x, W_enc.T, b_enc.reshape(1, H), W_lin0.T, W_self0.T,
        b_self0.reshape(1, H))
    agg0 = _sc_segment_sum(m0, e3, H // 2, jnp.float32,
                           feature_split=True, nbuf=4)
    s8, self1b, cnt0 = _tc_spike_mid(
        agg0, self0b, W_self1.T, b_self1.reshape(1, H))
    s_pack = jax.lax.bitcast_convert_type(
        s8.reshape(NPAD, H // 4, 4), jnp.int32)
    cnt1p = _sc_segment_sum(s_pack, e3, H // 4, jnp.int32,
                            feature_split=False, nbuf=6)
    perm = [4 * j + k for k in range(4) for j in range(H // 4)]
    Wl1Tp = W_lin1.T[jnp.array(perm), :]
    hp, npr, cnt1, _gf, gl = _tc_heads(
        cnt1p, self1b, Wl1Tp, Wn1.T, bn1.reshape(1, H // 2), Wn2,
        bn2.reshape(1, 1), Wc1.T, bc1.reshape(1, H // 2),
        jnp.pad(Wc2.T, ((0, 0), (0, H - 2))),
        jnp.pad(bc2, (0, H - 2)).reshape(1, H))

    return (gl[:, :2], npr, hp, cnt0[0, 0], cnt1[0, 0])
